# baseline (speedup 1.0000x reference)
import functools

import jax
import jax.numpy as jnp
from jax import lax
from jax.experimental import pallas as pl
from jax.experimental.pallas import tpu as pltpu
from jax.experimental.pallas import tpu_sc as plsc

E = 16
OBS_DIM = 768
ACT_DIM = 256
H1 = 256
H2 = 256
B = 8192
T = 256
PAD = B + E * T
NT = PAD // T
NW = 32
CHUNK = B // NW
SUB = 32
NSUB = CHUNK // SUB
OUTW = 128

_mesh = plsc.VectorSubcoreMesh(core_axis_name="c", subcore_axis_name="s",
                               num_cores=2, num_subcores=16)
_sc_params = pltpu.CompilerParams(needs_layout_passes=False)


def _wid():
    return lax.axis_index("s") * 2 + lax.axis_index("c")


@functools.partial(
    pl.kernel, mesh=_mesh, compiler_params=_sc_params,
    out_type=jax.ShapeDtypeStruct((NW, E), jnp.int32),
    scratch_types=[pltpu.VMEM((CHUNK,), jnp.int32),
                   pltpu.VMEM((E,), jnp.int32)])
def _hist_kernel(idx_hbm, hist_hbm, idxc, histv):
    wid = _wid()
    base = wid * CHUNK
    pltpu.sync_copy(idx_hbm.at[pl.ds(base, CHUNK)], idxc)
    lanes = lax.iota(jnp.int32, 16)
    hist = jnp.zeros((16,), jnp.int32)
    for k in range(CHUNK // 16):
        v = idxc[pl.ds(k * 16, 16)]
        for e in range(E):
            cnt = jnp.sum(jnp.where(v == e, 1, 0))
            hist = hist + jnp.where(lanes == e, cnt, 0)
    histv[...] = hist
    pltpu.sync_copy(histv, hist_hbm.at[wid])


@functools.partial(
    pl.kernel, mesh=_mesh, compiler_params=_sc_params,
    out_type=[jax.ShapeDtypeStruct((B,), jnp.int32),
              jax.ShapeDtypeStruct((PAD, OBS_DIM), jnp.float32),
              jax.ShapeDtypeStruct((PAD, ACT_DIM), jnp.float32),
              jax.ShapeDtypeStruct((NT,), jnp.int32)],
    scratch_types=[pltpu.VMEM((CHUNK,), jnp.int32),
                   pltpu.VMEM((NW, E), jnp.int32),
                   pltpu.VMEM((16,), jnp.int32),
                   pltpu.VMEM((NSUB, SUB), jnp.int32),
                   pltpu.VMEM((2, SUB, OBS_DIM), jnp.float32),
                   pltpu.VMEM((2, SUB, ACT_DIM), jnp.float32),
                   pltpu.VMEM((NT,), jnp.int32),
                   pltpu.SemaphoreType.DMA,
                   pltpu.SemaphoreType.DMA,
                   pltpu.SemaphoreType.DMA,
                   pltpu.SemaphoreType.DMA,
                   pltpu.SemaphoreType.DMA,
                   pltpu.SemaphoreType.DMA,
                   pltpu.SemaphoreType.DMA,
                   pltpu.SemaphoreType.DMA])
def _route_kernel(idx_hbm, hist_hbm, obs_hbm, act_hbm,
                  dest_hbm, xo_hbm, xa_hbm, te_hbm,
                  idxc, histv, cur, posb, obsb, actb, tebuf,
                  sro0, sro1, sra0, sra1, swo0, swo1, swa0, swa1):
    wid = _wid()
    base = wid * CHUNK
    sro = (sro0, sro1)
    sra = (sra0, sra1)
    swo = (swo0, swo1)
    swa = (swa0, swa1)
    pltpu.sync_copy(idx_hbm.at[pl.ds(base, CHUNK)], idxc)
    pltpu.sync_copy(hist_hbm, histv)
    lanes = lax.iota(jnp.int32, 16)

    total = jnp.zeros((16,), jnp.int32)
    start = jnp.zeros((16,), jnp.int32)
    for w in range(NW):
        h_w = histv[w]
        total = total + h_w
        start = start + jnp.where(jnp.int32(w) < wid, h_w, 0)
    padded = ((total + (T - 1)) >> 8) << 8
    cum = plsc.cumsum(padded)
    gbase = cum - padded
    start = start + gbase
    cur[...] = start

    for k in range(CHUNK // 16):
        v = idxc[pl.ds(k * 16, 16)]
        r = jnp.zeros((16,), jnp.int32)
        histu = jnp.zeros((16,), jnp.int32)
        for e in range(E):
            m = v == e
            c = plsc.cumsum(jnp.where(m, 1, 0))
            r = jnp.where(m, c - 1, r)
            cnt = jnp.sum(jnp.where(m, 1, 0))
            histu = histu + jnp.where(lanes == e, cnt, 0)
        kv = k // (SUB // 16)
        ks = k % (SUB // 16)
        pos = plsc.load_gather(cur, [v]) + r
        posb[kv, pl.ds(ks * 16, 16)] = pos
        cur[...] = cur[...] + histu

    for j in range(NSUB):
        pltpu.sync_copy(posb.at[j], dest_hbm.at[pl.ds(base + j * SUB, SUB)])

    h_ro = [None] * NSUB
    h_ra = [None] * NSUB
    h_wo = [None] * NSUB
    h_wa = [None] * NSUB

    def _read(j):
        pb = j % 2
        h_ro[j] = pltpu.async_copy(
            obs_hbm.at[pl.ds(base + j * SUB, SUB)], obsb.at[pb], sro[pb])
        h_ra[j] = pltpu.async_copy(
            act_hbm.at[pl.ds(base + j * SUB, SUB)], actb.at[pb], sra[pb])

    _read(0)
    for j in range(NSUB):
        pb = j % 2
        if j >= 1:
            h_wo[j - 1].wait()
            h_wa[j - 1].wait()
        if j + 1 < NSUB:
            _read(j + 1)
        h_ro[j].wait()
        h_ra[j].wait()
        h_wo[j] = pltpu.async_copy(obsb.at[pb], xo_hbm.at[posb.at[j]], swo[pb])
        h_wa[j] = pltpu.async_copy(actb.at[pb], xa_hbm.at[posb.at[j]], swa[pb])
    h_wo[NSUB - 1].wait()
    h_wa[NSUB - 1].wait()

    @pl.when(wid == 0)
    def _():
        ntiles = padded >> 8
        tlo = gbase >> 8
        for tv in range(NT // 16):
            tvec = lax.iota(jnp.int32, 16) + tv * 16
            acc = jnp.zeros((16,), jnp.int32)
            for e in range(E):
                lo = jnp.sum(jnp.where(lanes == e, tlo, 0))
                hi = lo + jnp.sum(jnp.where(lanes == e, ntiles, 0))
                acc = jnp.where((tvec >= lo) & (tvec < hi), e, acc)
            tebuf[pl.ds(tv * 16, 16)] = acc
        pltpu.sync_copy(tebuf, te_hbm)


def _mlp_body(te_ref, xo_ref, xa_ref, w1o_ref, w1a_ref, b1_ref, w2_ref,
              b2_ref, w3_ref, b3_ref, out_ref):
    bf = jnp.bfloat16
    e = te_ref[pl.program_id(0)]
    h = (jnp.dot(xo_ref[...].astype(bf), w1o_ref[e],
                 preferred_element_type=jnp.float32)
         + jnp.dot(xa_ref[...].astype(bf), w1a_ref[e],
                   preferred_element_type=jnp.float32)
         + b1_ref[e])
    h = jnp.maximum(h, 0.0)
    h = jnp.dot(h.astype(bf), w2_ref[e],
                preferred_element_type=jnp.float32) + b2_ref[e]
    h = jnp.maximum(h, 0.0)
    s = jnp.dot(h.astype(bf), w3_ref[e],
                preferred_element_type=jnp.float32) + b3_ref[e]
    out_ref[...] = jax.nn.sigmoid(s)


@functools.partial(
    pl.kernel, mesh=_mesh, compiler_params=_sc_params,
    out_type=jax.ShapeDtypeStruct((B,), jnp.float32),
    scratch_types=[pltpu.VMEM((CHUNK,), jnp.int32),
                   pltpu.VMEM((CHUNK // 4, OUTW), jnp.float32),
                   pltpu.VMEM((CHUNK,), jnp.float32),
                   pltpu.SemaphoreType.DMA])
def _out_gather_kernel(sig_hbm, dest_hbm, out_hbm, destb, rowsb, outb, sem):
    wid = _wid()
    base = wid * CHUNK
    gsub = CHUNK // 4
    zeros16 = jnp.zeros((16,), jnp.int32)
    pltpu.sync_copy(dest_hbm.at[pl.ds(base, CHUNK)], destb)
    for j in range(4):
        pltpu.async_copy(
            sig_hbm.at[destb.at[pl.ds(j * gsub, gsub)]], rowsb, sem).wait()
        for k in range(gsub // 16):
            rid = lax.iota(jnp.int32, 16) + k * 16
            outb[pl.ds(j * gsub + k * 16, 16)] = plsc.load_gather(
                rowsb, [rid, zeros16])
    pltpu.sync_copy(outb, out_hbm.at[pl.ds(base, CHUNK)])


def kernel(observation, action, skill_idx, W1, b1, W2, b2, W3, b3):
    bf = jnp.bfloat16
    idx = skill_idx.astype(jnp.int32)
    W1o = W1[:, :OBS_DIM, :].astype(bf)
    W1a = W1[:, OBS_DIM:, :].astype(bf)
    W3p = jnp.pad(W3, ((0, 0), (0, 0), (0, OUTW - 1))).astype(bf)
    b1r = b1[:, None, :]
    b2r = b2[:, None, :]
    b3p = jnp.pad(b3, ((0, 0), (0, OUTW - 1)))[:, None, :]

    hist = _hist_kernel(idx)
    dest, xo_g, xa_g, te = _route_kernel(idx, hist, observation, action)

    grid_spec = pltpu.PrefetchScalarGridSpec(
        num_scalar_prefetch=1,
        grid=(NT,),
        in_specs=[
            pl.BlockSpec((T, OBS_DIM), lambda t, te_r: (t, 0)),
            pl.BlockSpec((T, ACT_DIM), lambda t, te_r: (t, 0)),
            pl.BlockSpec((E, OBS_DIM, H1), lambda t, te_r: (0, 0, 0)),
            pl.BlockSpec((E, ACT_DIM, H1), lambda t, te_r: (0, 0, 0)),
            pl.BlockSpec((E, 1, H1), lambda t, te_r: (0, 0, 0)),
            pl.BlockSpec((E, H1, H2), lambda t, te_r: (0, 0, 0)),
            pl.BlockSpec((E, 1, H2), lambda t, te_r: (0, 0, 0)),
            pl.BlockSpec((E, H2, OUTW), lambda t, te_r: (0, 0, 0)),
            pl.BlockSpec((E, 1, OUTW), lambda t, te_r: (0, 0, 0)),
        ],
        out_specs=pl.BlockSpec((T, OUTW), lambda t, te_r: (t, 0)),
    )
    sig = pl.pallas_call(
        _mlp_body,
        grid_spec=grid_spec,
        out_shape=jax.ShapeDtypeStruct((PAD, OUTW), jnp.float32),
        compiler_params=pltpu.CompilerParams(
            dimension_semantics=("parallel",)),
    )(te, xo_g, xa_g, W1o, W1a, b1r, W2.astype(bf), b2r, W3p, b3p)

    out = _out_gather_kernel(sig, dest)
    return out.reshape(B, 1)


# speedup vs baseline: 1.1014x; 1.1014x over previous
import functools

import jax
import jax.numpy as jnp
from jax import lax
from jax.experimental import pallas as pl
from jax.experimental.pallas import tpu as pltpu
from jax.experimental.pallas import tpu_sc as plsc

E = 16
OBS_DIM = 768
ACT_DIM = 256
H1 = 256
H2 = 256
B = 8192
A = 128
PAD = B + E * A
NT = PAD // A
T = 512
SPT = T // A
NW = 32
CHUNK = B // NW
SUB = 32
NSUB = CHUNK // SUB
OUTW = 128

_mesh = plsc.VectorSubcoreMesh(core_axis_name="c", subcore_axis_name="s",
                               num_cores=2, num_subcores=16)
_sc_params = pltpu.CompilerParams(needs_layout_passes=False)


def _wid():
    return lax.axis_index("s") * 2 + lax.axis_index("c")


@functools.partial(
    pl.kernel, mesh=_mesh, compiler_params=_sc_params,
    out_type=jax.ShapeDtypeStruct((NW, E), jnp.int32),
    scratch_types=[pltpu.VMEM((CHUNK,), jnp.int32),
                   pltpu.VMEM((E,), jnp.int32)])
def _hist_kernel(idx_hbm, hist_hbm, idxc, histv):
    wid = _wid()
    base = wid * CHUNK
    pltpu.sync_copy(idx_hbm.at[pl.ds(base, CHUNK)], idxc)
    lanes = lax.iota(jnp.int32, 16)
    hist = jnp.zeros((16,), jnp.int32)
    for k in range(CHUNK // 16):
        v = idxc[pl.ds(k * 16, 16)]
        for e in range(E):
            cnt = jnp.sum(jnp.where(v == e, 1, 0))
            hist = hist + jnp.where(lanes == e, cnt, 0)
    histv[...] = hist
    pltpu.sync_copy(histv, hist_hbm.at[wid])


@functools.partial(
    pl.kernel, mesh=_mesh, compiler_params=_sc_params,
    out_type=[jax.ShapeDtypeStruct((B,), jnp.int32),
              jax.ShapeDtypeStruct((PAD, OBS_DIM), jnp.float32),
              jax.ShapeDtypeStruct((PAD, ACT_DIM), jnp.float32),
              jax.ShapeDtypeStruct((NT,), jnp.int32)],
    scratch_types=[pltpu.VMEM((CHUNK,), jnp.int32),
                   pltpu.VMEM((NW, E), jnp.int32),
                   pltpu.VMEM((16,), jnp.int32),
                   pltpu.VMEM((NSUB, SUB), jnp.int32),
                   pltpu.VMEM((2, SUB, OBS_DIM), jnp.float32),
                   pltpu.VMEM((2, SUB, ACT_DIM), jnp.float32),
                   pltpu.VMEM((NT,), jnp.int32),
                   pltpu.SemaphoreType.DMA,
                   pltpu.SemaphoreType.DMA,
                   pltpu.SemaphoreType.DMA,
                   pltpu.SemaphoreType.DMA,
                   pltpu.SemaphoreType.DMA,
                   pltpu.SemaphoreType.DMA,
                   pltpu.SemaphoreType.DMA,
                   pltpu.SemaphoreType.DMA])
def _route_kernel(idx_hbm, hist_hbm, obs_hbm, act_hbm,
                  dest_hbm, xo_hbm, xa_hbm, te_hbm,
                  idxc, histv, cur, posb, obsb, actb, tebuf,
                  sro0, sro1, sra0, sra1, swo0, swo1, swa0, swa1):
    wid = _wid()
    base = wid * CHUNK
    sro = (sro0, sro1)
    sra = (sra0, sra1)
    swo = (swo0, swo1)
    swa = (swa0, swa1)
    pltpu.sync_copy(idx_hbm.at[pl.ds(base, CHUNK)], idxc)
    pltpu.sync_copy(hist_hbm, histv)
    lanes = lax.iota(jnp.int32, 16)

    total = jnp.zeros((16,), jnp.int32)
    start = jnp.zeros((16,), jnp.int32)
    for w in range(NW):
        h_w = histv[w]
        total = total + h_w
        start = start + jnp.where(jnp.int32(w) < wid, h_w, 0)
    padded = ((total + (A - 1)) >> 7) << 7
    cum = plsc.cumsum(padded)
    gbase = cum - padded
    start = start + gbase
    cur[...] = start

    for k in range(CHUNK // 16):
        v = idxc[pl.ds(k * 16, 16)]
        r = jnp.zeros((16,), jnp.int32)
        histu = jnp.zeros((16,), jnp.int32)
        for e in range(E):
            m = v == e
            c = plsc.cumsum(jnp.where(m, 1, 0))
            r = jnp.where(m, c - 1, r)
            cnt = jnp.sum(jnp.where(m, 1, 0))
            histu = histu + jnp.where(lanes == e, cnt, 0)
        kv = k // (SUB // 16)
        ks = k % (SUB // 16)
        pos = plsc.load_gather(cur, [v]) + r
        posb[kv, pl.ds(ks * 16, 16)] = pos
        cur[...] = cur[...] + histu

    for j in range(NSUB):
        pltpu.sync_copy(posb.at[j], dest_hbm.at[pl.ds(base + j * SUB, SUB)])

    h_ro = [None] * NSUB
    h_ra = [None] * NSUB
    h_wo = [None] * NSUB
    h_wa = [None] * NSUB

    def _read(j):
        pb = j % 2
        h_ro[j] = pltpu.async_copy(
            obs_hbm.at[pl.ds(base + j * SUB, SUB)], obsb.at[pb], sro[pb])
        h_ra[j] = pltpu.async_copy(
            act_hbm.at[pl.ds(base + j * SUB, SUB)], actb.at[pb], sra[pb])

    _read(0)
    for j in range(NSUB):
        pb = j % 2
        if j >= 1:
            h_wo[j - 1].wait()
            h_wa[j - 1].wait()
        if j + 1 < NSUB:
            _read(j + 1)
        h_ro[j].wait()
        h_ra[j].wait()
        h_wo[j] = pltpu.async_copy(obsb.at[pb], xo_hbm.at[posb.at[j]], swo[pb])
        h_wa[j] = pltpu.async_copy(actb.at[pb], xa_hbm.at[posb.at[j]], swa[pb])
    h_wo[NSUB - 1].wait()
    h_wa[NSUB - 1].wait()

    @pl.when(wid == 0)
    def _():
        ntiles = padded >> 7
        tlo = gbase >> 7
        for tv in range(NT // 16):
            tvec = lax.iota(jnp.int32, 16) + tv * 16
            acc = jnp.zeros((16,), jnp.int32)
            for e in range(E):
                lo = jnp.sum(jnp.where(lanes == e, tlo, 0))
                hi = lo + jnp.sum(jnp.where(lanes == e, ntiles, 0))
                acc = jnp.where((tvec >= lo) & (tvec < hi), e, acc)
            tebuf[pl.ds(tv * 16, 16)] = acc
        pltpu.sync_copy(tebuf, te_hbm)


def _mlp_body(te_ref, xo_ref, xa_ref, w1o_ref, w1a_ref, b1_ref, w2_ref,
              b2_ref, w3_ref, b3_ref, out_ref):
    bf = jnp.bfloat16
    t = pl.program_id(0)
    for s in range(SPT):
        e = te_ref[t * SPT + s]
        rows = pl.ds(s * A, A)
        h = (jnp.dot(xo_ref[rows, :].astype(bf), w1o_ref[e],
                     preferred_element_type=jnp.float32)
             + jnp.dot(xa_ref[rows, :].astype(bf), w1a_ref[e],
                       preferred_element_type=jnp.float32)
             + b1_ref[e])
        h = jnp.maximum(h, 0.0)
        h = jnp.dot(h.astype(bf), w2_ref[e],
                    preferred_element_type=jnp.float32) + b2_ref[e]
        h = jnp.maximum(h, 0.0)
        sc = jnp.dot(h.astype(bf), w3_ref[e],
                     preferred_element_type=jnp.float32) + b3_ref[e]
        out_ref[rows, :] = jax.nn.sigmoid(sc)


@functools.partial(
    pl.kernel, mesh=_mesh, compiler_params=_sc_params,
    out_type=jax.ShapeDtypeStruct((B,), jnp.float32),
    scratch_types=[pltpu.VMEM((CHUNK,), jnp.int32),
                   pltpu.VMEM((CHUNK // 4, OUTW), jnp.float32),
                   pltpu.VMEM((CHUNK,), jnp.float32),
                   pltpu.SemaphoreType.DMA])
def _out_gather_kernel(sig_hbm, dest_hbm, out_hbm, destb, rowsb, outb, sem):
    wid = _wid()
    base = wid * CHUNK
    gsub = CHUNK // 4
    zeros16 = jnp.zeros((16,), jnp.int32)
    pltpu.sync_copy(dest_hbm.at[pl.ds(base, CHUNK)], destb)
    for j in range(4):
        pltpu.async_copy(
            sig_hbm.at[destb.at[pl.ds(j * gsub, gsub)]], rowsb, sem).wait()
        for k in range(gsub // 16):
            rid = lax.iota(jnp.int32, 16) + k * 16
            outb[pl.ds(j * gsub + k * 16, 16)] = plsc.load_gather(
                rowsb, [rid, zeros16])
    pltpu.sync_copy(outb, out_hbm.at[pl.ds(base, CHUNK)])


def kernel(observation, action, skill_idx, W1, b1, W2, b2, W3, b3):
    bf = jnp.bfloat16
    idx = skill_idx.astype(jnp.int32)
    W1o = W1[:, :OBS_DIM, :].astype(bf)
    W1a = W1[:, OBS_DIM:, :].astype(bf)
    W3p = jnp.pad(W3, ((0, 0), (0, 0), (0, OUTW - 1))).astype(bf)
    b1r = b1[:, None, :]
    b2r = b2[:, None, :]
    b3p = jnp.pad(b3, ((0, 0), (0, OUTW - 1)))[:, None, :]

    hist = _hist_kernel(idx)
    dest, xo_g, xa_g, te = _route_kernel(idx, hist, observation, action)

    grid_spec = pltpu.PrefetchScalarGridSpec(
        num_scalar_prefetch=1,
        grid=(PAD // T,),
        in_specs=[
            pl.BlockSpec((T, OBS_DIM), lambda t, te_r: (t, 0)),
            pl.BlockSpec((T, ACT_DIM), lambda t, te_r: (t, 0)),
            pl.BlockSpec((E, OBS_DIM, H1), lambda t, te_r: (0, 0, 0)),
            pl.BlockSpec((E, ACT_DIM, H1), lambda t, te_r: (0, 0, 0)),
            pl.BlockSpec((E, 1, H1), lambda t, te_r: (0, 0, 0)),
            pl.BlockSpec((E, H1, H2), lambda t, te_r: (0, 0, 0)),
            pl.BlockSpec((E, 1, H2), lambda t, te_r: (0, 0, 0)),
            pl.BlockSpec((E, H2, OUTW), lambda t, te_r: (0, 0, 0)),
            pl.BlockSpec((E, 1, OUTW), lambda t, te_r: (0, 0, 0)),
        ],
        out_specs=pl.BlockSpec((T, OUTW), lambda t, te_r: (t, 0)),
    )
    sig = pl.pallas_call(
        _mlp_body,
        grid_spec=grid_spec,
        out_shape=jax.ShapeDtypeStruct((PAD, OUTW), jnp.float32),
        compiler_params=pltpu.CompilerParams(
            dimension_semantics=("parallel",)),
    )(te, xo_g, xa_g, W1o, W1a, b1r, W2.astype(bf), b2r, W3p, b3p)

    out = _out_gather_kernel(sig, dest)
    return out.reshape(B, 1)


# speedup vs baseline: 1.1179x; 1.0150x over previous
import functools

import jax
import jax.numpy as jnp
from jax import lax
from jax.experimental import pallas as pl
from jax.experimental.pallas import tpu as pltpu
from jax.experimental.pallas import tpu_sc as plsc

E = 16
OBS_DIM = 768
ACT_DIM = 256
H1 = 256
H2 = 256
B = 8192
A = 128
PAD = B + E * A
NT = PAD // A
T = 1024
SPT = T // A
NW = 32
CHUNK = B // NW
SUB = 32
NSUB = CHUNK // SUB
OUTW = 128

_mesh = plsc.VectorSubcoreMesh(core_axis_name="c", subcore_axis_name="s",
                               num_cores=2, num_subcores=16)
_sc_params = pltpu.CompilerParams(needs_layout_passes=False)


def _wid():
    return lax.axis_index("s") * 2 + lax.axis_index("c")


@functools.partial(
    pl.kernel, mesh=_mesh, compiler_params=_sc_params,
    out_type=jax.ShapeDtypeStruct((NW, E), jnp.int32),
    scratch_types=[pltpu.VMEM((CHUNK,), jnp.int32),
                   pltpu.VMEM((E,), jnp.int32)])
def _hist_kernel(idx_hbm, hist_hbm, idxc, histv):
    wid = _wid()
    base = wid * CHUNK
    pltpu.sync_copy(idx_hbm.at[pl.ds(base, CHUNK)], idxc)
    lanes = lax.iota(jnp.int32, 16)
    hist = jnp.zeros((16,), jnp.int32)
    for k in range(CHUNK // 16):
        v = idxc[pl.ds(k * 16, 16)]
        for e in range(E):
            cnt = jnp.sum(jnp.where(v == e, 1, 0))
            hist = hist + jnp.where(lanes == e, cnt, 0)
    histv[...] = hist
    pltpu.sync_copy(histv, hist_hbm.at[wid])


@functools.partial(
    pl.kernel, mesh=_mesh, compiler_params=_sc_params,
    out_type=[jax.ShapeDtypeStruct((B,), jnp.int32),
              jax.ShapeDtypeStruct((PAD, OBS_DIM), jnp.float32),
              jax.ShapeDtypeStruct((PAD, ACT_DIM), jnp.float32),
              jax.ShapeDtypeStruct((NT,), jnp.int32)],
    scratch_types=[pltpu.VMEM((CHUNK,), jnp.int32),
                   pltpu.VMEM((NW, E), jnp.int32),
                   pltpu.VMEM((16,), jnp.int32),
                   pltpu.VMEM((NSUB, SUB), jnp.int32),
                   pltpu.VMEM((2, SUB, OBS_DIM), jnp.float32),
                   pltpu.VMEM((2, SUB, ACT_DIM), jnp.float32),
                   pltpu.VMEM((NT,), jnp.int32),
                   pltpu.SemaphoreType.DMA,
                   pltpu.SemaphoreType.DMA,
                   pltpu.SemaphoreType.DMA,
                   pltpu.SemaphoreType.DMA,
                   pltpu.SemaphoreType.DMA,
                   pltpu.SemaphoreType.DMA,
                   pltpu.SemaphoreType.DMA,
                   pltpu.SemaphoreType.DMA])
def _route_kernel(idx_hbm, hist_hbm, obs_hbm, act_hbm,
                  dest_hbm, xo_hbm, xa_hbm, te_hbm,
                  idxc, histv, cur, posb, obsb, actb, tebuf,
                  sro0, sro1, sra0, sra1, swo0, swo1, swa0, swa1):
    wid = _wid()
    base = wid * CHUNK
    sro = (sro0, sro1)
    sra = (sra0, sra1)
    swo = (swo0, swo1)
    swa = (swa0, swa1)
    pltpu.sync_copy(idx_hbm.at[pl.ds(base, CHUNK)], idxc)
    pltpu.sync_copy(hist_hbm, histv)
    lanes = lax.iota(jnp.int32, 16)

    total = jnp.zeros((16,), jnp.int32)
    start = jnp.zeros((16,), jnp.int32)
    for w in range(NW):
        h_w = histv[w]
        total = total + h_w
        start = start + jnp.where(jnp.int32(w) < wid, h_w, 0)
    padded = ((total + (A - 1)) >> 7) << 7
    cum = plsc.cumsum(padded)
    gbase = cum - padded
    start = start + gbase
    cur[...] = start

    for k in range(CHUNK // 16):
        v = idxc[pl.ds(k * 16, 16)]
        r = jnp.zeros((16,), jnp.int32)
        histu = jnp.zeros((16,), jnp.int32)
        for e in range(E):
            m = v == e
            c = plsc.cumsum(jnp.where(m, 1, 0))
            r = jnp.where(m, c - 1, r)
            cnt = jnp.sum(jnp.where(m, 1, 0))
            histu = histu + jnp.where(lanes == e, cnt, 0)
        kv = k // (SUB // 16)
        ks = k % (SUB // 16)
        pos = plsc.load_gather(cur, [v]) + r
        posb[kv, pl.ds(ks * 16, 16)] = pos
        cur[...] = cur[...] + histu

    for j in range(NSUB):
        pltpu.sync_copy(posb.at[j], dest_hbm.at[pl.ds(base + j * SUB, SUB)])

    h_ro = [None] * NSUB
    h_ra = [None] * NSUB
    h_wo = [None] * NSUB
    h_wa = [None] * NSUB

    def _read(j):
        pb = j % 2
        h_ro[j] = pltpu.async_copy(
            obs_hbm.at[pl.ds(base + j * SUB, SUB)], obsb.at[pb], sro[pb])
        h_ra[j] = pltpu.async_copy(
            act_hbm.at[pl.ds(base + j * SUB, SUB)], actb.at[pb], sra[pb])

    _read(0)
    for j in range(NSUB):
        pb = j % 2
        if j >= 1:
            h_wo[j - 1].wait()
            h_wa[j - 1].wait()
        if j + 1 < NSUB:
            _read(j + 1)
        h_ro[j].wait()
        h_ra[j].wait()
        h_wo[j] = pltpu.async_copy(obsb.at[pb], xo_hbm.at[posb.at[j]], swo[pb])
        h_wa[j] = pltpu.async_copy(actb.at[pb], xa_hbm.at[posb.at[j]], swa[pb])
    h_wo[NSUB - 1].wait()
    h_wa[NSUB - 1].wait()

    @pl.when(wid == 0)
    def _():
        ntiles = padded >> 7
        tlo = gbase >> 7
        for tv in range(NT // 16):
            tvec = lax.iota(jnp.int32, 16) + tv * 16
            acc = jnp.zeros((16,), jnp.int32)
            for e in range(E):
                lo = jnp.sum(jnp.where(lanes == e, tlo, 0))
                hi = lo + jnp.sum(jnp.where(lanes == e, ntiles, 0))
                acc = jnp.where((tvec >= lo) & (tvec < hi), e, acc)
            tebuf[pl.ds(tv * 16, 16)] = acc
        pltpu.sync_copy(tebuf, te_hbm)


def _mlp_body(te_ref, xo_ref, xa_ref, w1o_ref, w1a_ref, b1_ref, w2_ref,
              b2_ref, w3_ref, b3_ref, out_ref):
    bf = jnp.bfloat16
    t = pl.program_id(0)
    for s in range(SPT):
        e = te_ref[t * SPT + s]
        rows = pl.ds(s * A, A)
        h = (jnp.dot(xo_ref[rows, :].astype(bf), w1o_ref[e],
                     preferred_element_type=jnp.float32)
             + jnp.dot(xa_ref[rows, :].astype(bf), w1a_ref[e],
                       preferred_element_type=jnp.float32)
             + b1_ref[e])
        h = jnp.maximum(h, 0.0)
        h = jnp.dot(h.astype(bf), w2_ref[e],
                    preferred_element_type=jnp.float32) + b2_ref[e]
        h = jnp.maximum(h, 0.0)
        sc = jnp.dot(h.astype(bf), w3_ref[e],
                     preferred_element_type=jnp.float32) + b3_ref[e]
        out_ref[rows, :] = jax.nn.sigmoid(sc)


@functools.partial(
    pl.kernel, mesh=_mesh, compiler_params=_sc_params,
    out_type=jax.ShapeDtypeStruct((B,), jnp.float32),
    scratch_types=[pltpu.VMEM((CHUNK,), jnp.int32),
                   pltpu.VMEM((CHUNK // 4, OUTW), jnp.float32),
                   pltpu.VMEM((CHUNK,), jnp.float32),
                   pltpu.SemaphoreType.DMA])
def _out_gather_kernel(sig_hbm, dest_hbm, out_hbm, destb, rowsb, outb, sem):
    wid = _wid()
    base = wid * CHUNK
    gsub = CHUNK // 4
    zeros16 = jnp.zeros((16,), jnp.int32)
    pltpu.sync_copy(dest_hbm.at[pl.ds(base, CHUNK)], destb)
    for j in range(4):
        pltpu.async_copy(
            sig_hbm.at[destb.at[pl.ds(j * gsub, gsub)]], rowsb, sem).wait()
        for k in range(gsub // 16):
            rid = lax.iota(jnp.int32, 16) + k * 16
            outb[pl.ds(j * gsub + k * 16, 16)] = plsc.load_gather(
                rowsb, [rid, zeros16])
    pltpu.sync_copy(outb, out_hbm.at[pl.ds(base, CHUNK)])


def kernel(observation, action, skill_idx, W1, b1, W2, b2, W3, b3):
    bf = jnp.bfloat16
    idx = skill_idx.astype(jnp.int32)
    W1o = W1[:, :OBS_DIM, :].astype(bf)
    W1a = W1[:, OBS_DIM:, :].astype(bf)
    W3p = jnp.pad(W3, ((0, 0), (0, 0), (0, OUTW - 1))).astype(bf)
    b1r = b1[:, None, :]
    b2r = b2[:, None, :]
    b3p = jnp.pad(b3, ((0, 0), (0, OUTW - 1)))[:, None, :]

    hist = _hist_kernel(idx)
    dest, xo_g, xa_g, te = _route_kernel(idx, hist, observation, action)

    grid_spec = pltpu.PrefetchScalarGridSpec(
        num_scalar_prefetch=1,
        grid=(PAD // T,),
        in_specs=[
            pl.BlockSpec((T, OBS_DIM), lambda t, te_r: (t, 0)),
            pl.BlockSpec((T, ACT_DIM), lambda t, te_r: (t, 0)),
            pl.BlockSpec((E, OBS_DIM, H1), lambda t, te_r: (0, 0, 0)),
            pl.BlockSpec((E, ACT_DIM, H1), lambda t, te_r: (0, 0, 0)),
            pl.BlockSpec((E, 1, H1), lambda t, te_r: (0, 0, 0)),
            pl.BlockSpec((E, H1, H2), lambda t, te_r: (0, 0, 0)),
            pl.BlockSpec((E, 1, H2), lambda t, te_r: (0, 0, 0)),
            pl.BlockSpec((E, H2, OUTW), lambda t, te_r: (0, 0, 0)),
            pl.BlockSpec((E, 1, OUTW), lambda t, te_r: (0, 0, 0)),
        ],
        out_specs=pl.BlockSpec((T, OUTW), lambda t, te_r: (t, 0)),
    )
    sig = pl.pallas_call(
        _mlp_body,
        grid_spec=grid_spec,
        out_shape=jax.ShapeDtypeStruct((PAD, OUTW), jnp.float32),
        compiler_params=pltpu.CompilerParams(
            dimension_semantics=("parallel",)),
    )(te, xo_g, xa_g, W1o, W1a, b1r, W2.astype(bf), b2r, W3p, b3p)

    out = _out_gather_kernel(sig, dest)
    return out.reshape(B, 1)


# speedup vs baseline: 1.1466x; 1.0257x over previous
import functools

import jax
import jax.numpy as jnp
from jax import lax
from jax.experimental import pallas as pl
from jax.experimental.pallas import tpu as pltpu
from jax.experimental.pallas import tpu_sc as plsc

E = 16
OBS_DIM = 768
ACT_DIM = 256
H1 = 256
H2 = 256
B = 8192
A = 128
PAD = B + E * A
NT = PAD // A
T = 1024
SPT = T // A
NW = 32
CHUNK = B // NW
SUB = 32
NSUB = CHUNK // SUB
OUTW = 128

_mesh = plsc.VectorSubcoreMesh(core_axis_name="c", subcore_axis_name="s",
                               num_cores=2, num_subcores=16)
_sc_params = pltpu.CompilerParams(needs_layout_passes=False)


def _wid():
    return lax.axis_index("s") * 2 + lax.axis_index("c")


def _hist_tc_body(idx_ref, out_ref):
    m = idx_ref[...]
    lanes16 = lax.broadcasted_iota(jnp.int32, (1, E), 1)
    acc = jnp.zeros((NW, E), jnp.int32)
    for e in range(E):
        ce = jnp.sum((m == e).astype(jnp.int32), axis=1, keepdims=True)
        cw = jnp.sum(ce.reshape(NW, 2), axis=1, keepdims=True)
        acc = acc + cw * (lanes16 == e).astype(jnp.int32)
    out_ref[...] = acc


@functools.partial(
    pl.kernel, mesh=_mesh, compiler_params=_sc_params,
    out_type=jax.ShapeDtypeStruct((NW, E), jnp.int32),
    scratch_types=[pltpu.VMEM((CHUNK,), jnp.int32),
                   pltpu.VMEM((E,), jnp.int32)])
def _hist_kernel(idx_hbm, hist_hbm, idxc, histv):
    wid = _wid()
    base = wid * CHUNK
    pltpu.sync_copy(idx_hbm.at[pl.ds(base, CHUNK)], idxc)
    lanes = lax.iota(jnp.int32, 16)
    hist = jnp.zeros((16,), jnp.int32)
    for k in range(CHUNK // 16):
        v = idxc[pl.ds(k * 16, 16)]
        for e in range(E):
            cnt = jnp.sum(jnp.where(v == e, 1, 0))
            hist = hist + jnp.where(lanes == e, cnt, 0)
    histv[...] = hist
    pltpu.sync_copy(histv, hist_hbm.at[wid])


@functools.partial(
    pl.kernel, mesh=_mesh, compiler_params=_sc_params,
    out_type=[jax.ShapeDtypeStruct((B,), jnp.int32),
              jax.ShapeDtypeStruct((PAD, OBS_DIM), jnp.float32),
              jax.ShapeDtypeStruct((PAD, ACT_DIM), jnp.float32),
              jax.ShapeDtypeStruct((NT,), jnp.int32)],
    scratch_types=[pltpu.VMEM((CHUNK,), jnp.int32),
                   pltpu.VMEM((NW, E), jnp.int32),
                   pltpu.VMEM((16,), jnp.int32),
                   pltpu.VMEM((NSUB, SUB), jnp.int32),
                   pltpu.VMEM((2, SUB, OBS_DIM), jnp.float32),
                   pltpu.VMEM((2, SUB, ACT_DIM), jnp.float32),
                   pltpu.VMEM((NT,), jnp.int32),
                   pltpu.SemaphoreType.DMA,
                   pltpu.SemaphoreType.DMA,
                   pltpu.SemaphoreType.DMA,
                   pltpu.SemaphoreType.DMA,
                   pltpu.SemaphoreType.DMA,
                   pltpu.SemaphoreType.DMA,
                   pltpu.SemaphoreType.DMA,
                   pltpu.SemaphoreType.DMA])
def _route_kernel(idx_hbm, hist_hbm, obs_hbm, act_hbm,
                  dest_hbm, xo_hbm, xa_hbm, te_hbm,
                  idxc, histv, cur, posb, obsb, actb, tebuf,
                  sro0, sro1, sra0, sra1, swo0, swo1, swa0, swa1):
    wid = _wid()
    base = wid * CHUNK
    sro = (sro0, sro1)
    sra = (sra0, sra1)
    swo = (swo0, swo1)
    swa = (swa0, swa1)
    pltpu.sync_copy(idx_hbm.at[pl.ds(base, CHUNK)], idxc)
    pltpu.sync_copy(hist_hbm, histv)
    lanes = lax.iota(jnp.int32, 16)

    total = jnp.zeros((16,), jnp.int32)
    start = jnp.zeros((16,), jnp.int32)
    for w in range(NW):
        h_w = histv[w]
        total = total + h_w
        start = start + jnp.where(jnp.int32(w) < wid, h_w, 0)
    padded = ((total + (A - 1)) >> 7) << 7
    cum = plsc.cumsum(padded)
    gbase = cum - padded
    start = start + gbase
    cur[...] = start

    for k in range(CHUNK // 16):
        v = idxc[pl.ds(k * 16, 16)]
        r = jnp.zeros((16,), jnp.int32)
        histu = jnp.zeros((16,), jnp.int32)
        for e in range(E):
            m = v == e
            c = plsc.cumsum(jnp.where(m, 1, 0))
            r = jnp.where(m, c - 1, r)
            cnt = jnp.sum(jnp.where(m, 1, 0))
            histu = histu + jnp.where(lanes == e, cnt, 0)
        kv = k // (SUB // 16)
        ks = k % (SUB // 16)
        pos = plsc.load_gather(cur, [v]) + r
        posb[kv, pl.ds(ks * 16, 16)] = pos
        cur[...] = cur[...] + histu

    for j in range(NSUB):
        pltpu.sync_copy(posb.at[j], dest_hbm.at[pl.ds(base + j * SUB, SUB)])

    h_ro = [None] * NSUB
    h_ra = [None] * NSUB
    h_wo = [None] * NSUB
    h_wa = [None] * NSUB

    def _read(j):
        pb = j % 2
        h_ro[j] = pltpu.async_copy(
            obs_hbm.at[pl.ds(base + j * SUB, SUB)], obsb.at[pb], sro[pb])
        h_ra[j] = pltpu.async_copy(
            act_hbm.at[pl.ds(base + j * SUB, SUB)], actb.at[pb], sra[pb])

    _read(0)
    for j in range(NSUB):
        pb = j % 2
        if j >= 1:
            h_wo[j - 1].wait()
            h_wa[j - 1].wait()
        if j + 1 < NSUB:
            _read(j + 1)
        h_ro[j].wait()
        h_ra[j].wait()
        h_wo[j] = pltpu.async_copy(obsb.at[pb], xo_hbm.at[posb.at[j]], swo[pb])
        h_wa[j] = pltpu.async_copy(actb.at[pb], xa_hbm.at[posb.at[j]], swa[pb])
    h_wo[NSUB - 1].wait()
    h_wa[NSUB - 1].wait()

    @pl.when(wid == 0)
    def _():
        ntiles = padded >> 7
        tlo = gbase >> 7
        for tv in range(NT // 16):
            tvec = lax.iota(jnp.int32, 16) + tv * 16
            acc = jnp.zeros((16,), jnp.int32)
            for e in range(E):
                lo = jnp.sum(jnp.where(lanes == e, tlo, 0))
                hi = lo + jnp.sum(jnp.where(lanes == e, ntiles, 0))
                acc = jnp.where((tvec >= lo) & (tvec < hi), e, acc)
            tebuf[pl.ds(tv * 16, 16)] = acc
        pltpu.sync_copy(tebuf, te_hbm)


def _mlp_body(te_ref, xo_ref, xa_ref, w1o_ref, w1a_ref, b1_ref, w2_ref,
              b2_ref, w3_ref, b3_ref, out_ref):
    bf = jnp.bfloat16
    t = pl.program_id(0)
    for s in range(SPT):
        e = te_ref[t * SPT + s]
        rows = pl.ds(s * A, A)
        h = (jnp.dot(xo_ref[rows, :].astype(bf), w1o_ref[e],
                     preferred_element_type=jnp.float32)
             + jnp.dot(xa_ref[rows, :].astype(bf), w1a_ref[e],
                       preferred_element_type=jnp.float32)
             + b1_ref[e])
        h = jnp.maximum(h, 0.0)
        h = jnp.dot(h.astype(bf), w2_ref[e],
                    preferred_element_type=jnp.float32) + b2_ref[e]
        h = jnp.maximum(h, 0.0)
        sc = jnp.dot(h.astype(bf), w3_ref[e],
                     preferred_element_type=jnp.float32) + b3_ref[e]
        out_ref[rows, :] = jax.nn.sigmoid(sc)


@functools.partial(
    pl.kernel, mesh=_mesh, compiler_params=_sc_params,
    out_type=jax.ShapeDtypeStruct((B,), jnp.float32),
    scratch_types=[pltpu.VMEM((CHUNK,), jnp.int32),
                   pltpu.VMEM((CHUNK // 4, OUTW), jnp.float32),
                   pltpu.VMEM((CHUNK,), jnp.float32),
                   pltpu.SemaphoreType.DMA])
def _out_gather_kernel(sig_hbm, dest_hbm, out_hbm, destb, rowsb, outb, sem):
    wid = _wid()
    base = wid * CHUNK
    gsub = CHUNK // 4
    zeros16 = jnp.zeros((16,), jnp.int32)
    pltpu.sync_copy(dest_hbm.at[pl.ds(base, CHUNK)], destb)
    for j in range(4):
        pltpu.async_copy(
            sig_hbm.at[destb.at[pl.ds(j * gsub, gsub)]], rowsb, sem).wait()
        for k in range(gsub // 16):
            rid = lax.iota(jnp.int32, 16) + k * 16
            outb[pl.ds(j * gsub + k * 16, 16)] = plsc.load_gather(
                rowsb, [rid, zeros16])
    pltpu.sync_copy(outb, out_hbm.at[pl.ds(base, CHUNK)])


def kernel(observation, action, skill_idx, W1, b1, W2, b2, W3, b3):
    bf = jnp.bfloat16
    idx = skill_idx.astype(jnp.int32)
    W1o = W1[:, :OBS_DIM, :].astype(bf)
    W1a = W1[:, OBS_DIM:, :].astype(bf)
    W3p = jnp.pad(W3, ((0, 0), (0, 0), (0, OUTW - 1))).astype(bf)
    b1r = b1[:, None, :]
    b2r = b2[:, None, :]
    b3p = jnp.pad(b3, ((0, 0), (0, OUTW - 1)))[:, None, :]

    hist = pl.pallas_call(
        _hist_tc_body,
        out_shape=jax.ShapeDtypeStruct((NW, E), jnp.int32),
    )(idx.reshape(NW * 2, 128))
    dest, xo_g, xa_g, te = _route_kernel(idx, hist, observation, action)

    grid_spec = pltpu.PrefetchScalarGridSpec(
        num_scalar_prefetch=1,
        grid=(PAD // T,),
        in_specs=[
            pl.BlockSpec((T, OBS_DIM), lambda t, te_r: (t, 0)),
            pl.BlockSpec((T, ACT_DIM), lambda t, te_r: (t, 0)),
            pl.BlockSpec((E, OBS_DIM, H1), lambda t, te_r: (0, 0, 0)),
            pl.BlockSpec((E, ACT_DIM, H1), lambda t, te_r: (0, 0, 0)),
            pl.BlockSpec((E, 1, H1), lambda t, te_r: (0, 0, 0)),
            pl.BlockSpec((E, H1, H2), lambda t, te_r: (0, 0, 0)),
            pl.BlockSpec((E, 1, H2), lambda t, te_r: (0, 0, 0)),
            pl.BlockSpec((E, H2, OUTW), lambda t, te_r: (0, 0, 0)),
            pl.BlockSpec((E, 1, OUTW), lambda t, te_r: (0, 0, 0)),
        ],
        out_specs=pl.BlockSpec((T, OUTW), lambda t, te_r: (t, 0)),
    )
    sig = pl.pallas_call(
        _mlp_body,
        grid_spec=grid_spec,
        out_shape=jax.ShapeDtypeStruct((PAD, OUTW), jnp.float32),
        compiler_params=pltpu.CompilerParams(
            dimension_semantics=("parallel",)),
    )(te, xo_g, xa_g, W1o, W1a, b1r, W2.astype(bf), b2r, W3p, b3p)

    out = _out_gather_kernel(sig, dest)
    return out.reshape(B, 1)


# speedup vs baseline: 1.1633x; 1.0146x over previous
import functools

import jax
import jax.numpy as jnp
from jax import lax
from jax.experimental import pallas as pl
from jax.experimental.pallas import tpu as pltpu
from jax.experimental.pallas import tpu_sc as plsc

E = 16
OBS_DIM = 768
ACT_DIM = 256
H1 = 256
H2 = 256
B = 8192
A = 128
PAD = B + E * A
NT = PAD // A
T = 1024
SPT = T // A
NW = 32
CHUNK = B // NW
SUB = 32
NSUB = CHUNK // SUB
OUTW = 128

_mesh = plsc.VectorSubcoreMesh(core_axis_name="c", subcore_axis_name="s",
                               num_cores=2, num_subcores=16)
_sc_params = pltpu.CompilerParams(needs_layout_passes=False)


def _wid():
    return lax.axis_index("s") * 2 + lax.axis_index("c")


def _hist_tc_body(idx_ref, out_ref):
    m = idx_ref[...]
    lanes16 = lax.broadcasted_iota(jnp.int32, (1, E), 1)
    acc = jnp.zeros((NW, E), jnp.int32)
    for e in range(E):
        ce = jnp.sum((m == e).astype(jnp.int32), axis=1, keepdims=True)
        cw = jnp.sum(ce.reshape(NW, 2), axis=1, keepdims=True)
        acc = acc + cw * (lanes16 == e).astype(jnp.int32)
    out_ref[...] = acc


@functools.partial(
    pl.kernel, mesh=_mesh, compiler_params=_sc_params,
    out_type=jax.ShapeDtypeStruct((NW, E), jnp.int32),
    scratch_types=[pltpu.VMEM((CHUNK,), jnp.int32),
                   pltpu.VMEM((E,), jnp.int32)])
def _hist_kernel(idx_hbm, hist_hbm, idxc, histv):
    wid = _wid()
    base = wid * CHUNK
    pltpu.sync_copy(idx_hbm.at[pl.ds(base, CHUNK)], idxc)
    lanes = lax.iota(jnp.int32, 16)
    hist = jnp.zeros((16,), jnp.int32)
    for k in range(CHUNK // 16):
        v = idxc[pl.ds(k * 16, 16)]
        for e in range(E):
            cnt = jnp.sum(jnp.where(v == e, 1, 0))
            hist = hist + jnp.where(lanes == e, cnt, 0)
    histv[...] = hist
    pltpu.sync_copy(histv, hist_hbm.at[wid])


@functools.partial(
    pl.kernel, mesh=_mesh, compiler_params=_sc_params,
    out_type=[jax.ShapeDtypeStruct((B,), jnp.int32),
              jax.ShapeDtypeStruct((PAD, OBS_DIM), jnp.float32),
              jax.ShapeDtypeStruct((PAD, ACT_DIM), jnp.float32),
              jax.ShapeDtypeStruct((NT,), jnp.int32)],
    scratch_types=[pltpu.VMEM((CHUNK,), jnp.int32),
                   pltpu.VMEM((NW, E), jnp.int32),
                   pltpu.VMEM((16,), jnp.int32),
                   pltpu.VMEM((NSUB, SUB), jnp.int32),
                   pltpu.VMEM((3, SUB, OBS_DIM), jnp.float32),
                   pltpu.VMEM((3, SUB, ACT_DIM), jnp.float32),
                   pltpu.VMEM((NT,), jnp.int32),
                   pltpu.SemaphoreType.DMA,
                   pltpu.SemaphoreType.DMA,
                   pltpu.SemaphoreType.DMA,
                   pltpu.SemaphoreType.DMA,
                   pltpu.SemaphoreType.DMA,
                   pltpu.SemaphoreType.DMA,
                   pltpu.SemaphoreType.DMA,
                   pltpu.SemaphoreType.DMA,
                   pltpu.SemaphoreType.DMA,
                   pltpu.SemaphoreType.DMA])
def _route_kernel(idx_hbm, hist_hbm, obs_hbm, act_hbm,
                  dest_hbm, xo_hbm, xa_hbm, te_hbm,
                  idxc, histv, cur, posb, obsb, actb, tebuf,
                  sro0, sro1, sro2, sra0, sra1, sra2, swo0, swo1, swa0, swa1):
    wid = _wid()
    base = wid * CHUNK
    sro = (sro0, sro1, sro2)
    sra = (sra0, sra1, sra2)
    swo = (swo0, swo1)
    swa = (swa0, swa1)
    pltpu.sync_copy(idx_hbm.at[pl.ds(base, CHUNK)], idxc)

    h_ro = [None] * NSUB
    h_ra = [None] * NSUB
    h_wo = [None] * NSUB
    h_wa = [None] * NSUB

    def _read(j):
        pb = j % 3
        h_ro[j] = pltpu.async_copy(
            obs_hbm.at[pl.ds(base + j * SUB, SUB)], obsb.at[pb], sro[pb])
        h_ra[j] = pltpu.async_copy(
            act_hbm.at[pl.ds(base + j * SUB, SUB)], actb.at[pb], sra[pb])

    _read(0)
    _read(1)
    pltpu.sync_copy(hist_hbm, histv)
    lanes = lax.iota(jnp.int32, 16)

    total = jnp.zeros((16,), jnp.int32)
    start = jnp.zeros((16,), jnp.int32)
    for w in range(NW):
        h_w = histv[w]
        total = total + h_w
        start = start + jnp.where(jnp.int32(w) < wid, h_w, 0)
    padded = ((total + (A - 1)) >> 7) << 7
    cum = plsc.cumsum(padded)
    gbase = cum - padded
    start = start + gbase
    cur[...] = start

    for k in range(CHUNK // 16):
        v = idxc[pl.ds(k * 16, 16)]
        r = jnp.zeros((16,), jnp.int32)
        histu = jnp.zeros((16,), jnp.int32)
        for e in range(E):
            m = v == e
            c = plsc.cumsum(jnp.where(m, 1, 0))
            r = jnp.where(m, c - 1, r)
            cnt = jnp.sum(jnp.where(m, 1, 0))
            histu = histu + jnp.where(lanes == e, cnt, 0)
        kv = k // (SUB // 16)
        ks = k % (SUB // 16)
        pos = plsc.load_gather(cur, [v]) + r
        posb[kv, pl.ds(ks * 16, 16)] = pos
        cur[...] = cur[...] + histu

    for j in range(NSUB):
        pltpu.sync_copy(posb.at[j], dest_hbm.at[pl.ds(base + j * SUB, SUB)])

    for j in range(NSUB):
        pb = j % 3
        if j >= 1:
            h_wo[j - 1].wait()
            h_wa[j - 1].wait()
        if j + 2 < NSUB:
            _read(j + 2)
        h_ro[j].wait()
        h_ra[j].wait()
        h_wo[j] = pltpu.async_copy(obsb.at[pb], xo_hbm.at[posb.at[j]],
                                   swo[j % 2])
        h_wa[j] = pltpu.async_copy(actb.at[pb], xa_hbm.at[posb.at[j]],
                                   swa[j % 2])
    h_wo[NSUB - 1].wait()
    h_wa[NSUB - 1].wait()

    @pl.when(wid == 0)
    def _():
        ntiles = padded >> 7
        tlo = gbase >> 7
        for tv in range(NT // 16):
            tvec = lax.iota(jnp.int32, 16) + tv * 16
            acc = jnp.zeros((16,), jnp.int32)
            for e in range(E):
                lo = jnp.sum(jnp.where(lanes == e, tlo, 0))
                hi = lo + jnp.sum(jnp.where(lanes == e, ntiles, 0))
                acc = jnp.where((tvec >= lo) & (tvec < hi), e, acc)
            tebuf[pl.ds(tv * 16, 16)] = acc
        pltpu.sync_copy(tebuf, te_hbm)


def _mlp_body(te_ref, xo_ref, xa_ref, w1o_ref, w1a_ref, b1_ref, w2_ref,
              b2_ref, w3_ref, b3_ref, out_ref):
    bf = jnp.bfloat16
    t = pl.program_id(0)
    for s in range(SPT):
        e = te_ref[t * SPT + s]
        rows = pl.ds(s * A, A)
        h = (jnp.dot(xo_ref[rows, :].astype(bf), w1o_ref[e],
                     preferred_element_type=jnp.float32)
             + jnp.dot(xa_ref[rows, :].astype(bf), w1a_ref[e],
                       preferred_element_type=jnp.float32)
             + b1_ref[e])
        h = jnp.maximum(h, 0.0)
        h = jnp.dot(h.astype(bf), w2_ref[e],
                    preferred_element_type=jnp.float32) + b2_ref[e]
        h = jnp.maximum(h, 0.0)
        sc = jnp.dot(h.astype(bf), w3_ref[e],
                     preferred_element_type=jnp.float32) + b3_ref[e]
        out_ref[rows, :] = jax.nn.sigmoid(sc)


@functools.partial(
    pl.kernel, mesh=_mesh, compiler_params=_sc_params,
    out_type=jax.ShapeDtypeStruct((B,), jnp.float32),
    scratch_types=[pltpu.VMEM((CHUNK,), jnp.int32),
                   pltpu.VMEM((CHUNK // 4, OUTW), jnp.float32),
                   pltpu.VMEM((CHUNK,), jnp.float32),
                   pltpu.SemaphoreType.DMA])
def _out_gather_kernel(sig_hbm, dest_hbm, out_hbm, destb, rowsb, outb, sem):
    wid = _wid()
    base = wid * CHUNK
    gsub = CHUNK // 4
    zeros16 = jnp.zeros((16,), jnp.int32)
    pltpu.sync_copy(dest_hbm.at[pl.ds(base, CHUNK)], destb)
    for j in range(4):
        pltpu.async_copy(
            sig_hbm.at[destb.at[pl.ds(j * gsub, gsub)]], rowsb, sem).wait()
        for k in range(gsub // 16):
            rid = lax.iota(jnp.int32, 16) + k * 16
            outb[pl.ds(j * gsub + k * 16, 16)] = plsc.load_gather(
                rowsb, [rid, zeros16])
    pltpu.sync_copy(outb, out_hbm.at[pl.ds(base, CHUNK)])


def kernel(observation, action, skill_idx, W1, b1, W2, b2, W3, b3):
    bf = jnp.bfloat16
    idx = skill_idx.astype(jnp.int32)
    W1o = W1[:, :OBS_DIM, :].astype(bf)
    W1a = W1[:, OBS_DIM:, :].astype(bf)
    W3p = jnp.pad(W3, ((0, 0), (0, 0), (0, OUTW - 1))).astype(bf)
    b1r = b1[:, None, :]
    b2r = b2[:, None, :]
    b3p = jnp.pad(b3, ((0, 0), (0, OUTW - 1)))[:, None, :]

    hist = pl.pallas_call(
        _hist_tc_body,
        out_shape=jax.ShapeDtypeStruct((NW, E), jnp.int32),
    )(idx.reshape(NW * 2, 128))
    dest, xo_g, xa_g, te = _route_kernel(idx, hist, observation, action)

    grid_spec = pltpu.PrefetchScalarGridSpec(
        num_scalar_prefetch=1,
        grid=(PAD // T,),
        in_specs=[
            pl.BlockSpec((T, OBS_DIM), lambda t, te_r: (t, 0)),
            pl.BlockSpec((T, ACT_DIM), lambda t, te_r: (t, 0)),
            pl.BlockSpec((E, OBS_DIM, H1), lambda t, te_r: (0, 0, 0)),
            pl.BlockSpec((E, ACT_DIM, H1), lambda t, te_r: (0, 0, 0)),
            pl.BlockSpec((E, 1, H1), lambda t, te_r: (0, 0, 0)),
            pl.BlockSpec((E, H1, H2), lambda t, te_r: (0, 0, 0)),
            pl.BlockSpec((E, 1, H2), lambda t, te_r: (0, 0, 0)),
            pl.BlockSpec((E, H2, OUTW), lambda t, te_r: (0, 0, 0)),
            pl.BlockSpec((E, 1, OUTW), lambda t, te_r: (0, 0, 0)),
        ],
        out_specs=pl.BlockSpec((T, OUTW), lambda t, te_r: (t, 0)),
    )
    sig = pl.pallas_call(
        _mlp_body,
        grid_spec=grid_spec,
        out_shape=jax.ShapeDtypeStruct((PAD, OUTW), jnp.float32),
        compiler_params=pltpu.CompilerParams(
            dimension_semantics=("parallel",)),
    )(te, xo_g, xa_g, W1o, W1a, b1r, W2.astype(bf), b2r, W3p, b3p)

    out = _out_gather_kernel(sig, dest)
    return out.reshape(B, 1)
